# baseline (speedup 1.0000x reference)
import functools

import jax
import jax.numpy as jnp
from jax import lax
from jax.experimental import pallas as pl
from jax.experimental.pallas import tpu as pltpu
from jax.experimental.pallas import tpu_sc as plsc

_ROWS = 1000000
_COLS = 64
_NW = 32
_SPAN = 31248
_CHUNK = 496
_NCH = _SPAN // _CHUNK
_NBUF = 3
_GROUPS = _NCH // _NBUF
_TAIL = _ROWS - _NW * _SPAN


def _sc_body(x_hbm, o_hbm, buf, in_sems, out_sems):
    c = lax.axis_index("c")
    s = lax.axis_index("s")
    wid = s * 2 + c
    base = wid * _SPAN

    def in_copy(k, b):
        return pltpu.make_async_copy(
            x_hbm.at[pl.ds(base + k * _CHUNK, _CHUNK), :],
            buf.at[b],
            in_sems.at[b],
        )

    def out_copy(k, b):
        return pltpu.make_async_copy(
            buf.at[b],
            o_hbm.at[pl.ds(base + k * _CHUNK, _CHUNK), :],
            out_sems.at[b],
        )

    def group(g, carry):
        for t in range(_NBUF):
            k = g * _NBUF + t

            @pl.when(k >= _NBUF)
            def _():
                out_copy(k - _NBUF, t).wait()

            in_copy(k, t).start()

            @pl.when(k >= 1)
            def _():
                j = k - 1
                bj = (t - 1) % _NBUF
                in_copy(j, bj).wait()
                out_copy(j, bj).start()

        return carry

    lax.fori_loop(0, _GROUPS, group, 0)

    last = _NCH - 1
    in_copy(last, last % _NBUF).wait()
    out_copy(last, last % _NBUF).start()
    for t in range(_NBUF):
        k = _NCH - _NBUF + t
        out_copy(k, k % _NBUF).wait()

    @pl.when(wid == _NW - 1)
    def _():
        t0 = _NW * _SPAN
        pltpu.make_async_copy(
            x_hbm.at[pl.ds(t0, _TAIL), :],
            buf.at[0, pl.ds(0, _TAIL), :],
            in_sems.at[0],
        ).start()
        pltpu.make_async_copy(
            x_hbm.at[pl.ds(t0, _TAIL), :],
            buf.at[0, pl.ds(0, _TAIL), :],
            in_sems.at[0],
        ).wait()
        pltpu.make_async_copy(
            buf.at[0, pl.ds(0, _TAIL), :],
            o_hbm.at[pl.ds(t0, _TAIL), :],
            out_sems.at[0],
        ).start()
        pltpu.make_async_copy(
            buf.at[0, pl.ds(0, _TAIL), :],
            o_hbm.at[pl.ds(t0, _TAIL), :],
            out_sems.at[0],
        ).wait()


def kernel(table):
    mesh = plsc.VectorSubcoreMesh(core_axis_name="c", subcore_axis_name="s")
    f = functools.partial(
        pl.kernel,
        out_type=jax.ShapeDtypeStruct((_ROWS, _COLS), table.dtype),
        mesh=mesh,
        scratch_types=[
            pltpu.VMEM((_NBUF, _CHUNK, _COLS), table.dtype),
            pltpu.SemaphoreType.DMA((_NBUF,)),
            pltpu.SemaphoreType.DMA((_NBUF,)),
        ],
        compiler_params=pltpu.CompilerParams(use_tc_tiling_on_sc=False),
    )(_sc_body)
    return f(table)


# speedup vs baseline: 1.3848x; 1.3848x over previous
import jax
import jax.numpy as jnp
from jax.experimental import pallas as pl
from jax.experimental.pallas import tpu as pltpu

_OUTER = 125
_CHUNK = 64
_NBUF = 12
_LAG = 6


def _copy_body(x_ref, o_ref, buf, in_sems, out_sems):
    rows, cols = x_ref.shape
    inner = rows // _OUTER
    x3 = x_ref.reshape(_OUTER, inner, cols)
    o3 = o_ref.reshape(_OUTER, inner, cols)
    nch = inner // _CHUNK

    def in_copy(i, b):
        return pltpu.make_async_copy(
            x3.at[:, pl.ds(i * _CHUNK, _CHUNK), :],
            buf.at[b],
            in_sems.at[b],
        )

    def out_copy(i, b):
        return pltpu.make_async_copy(
            buf.at[b],
            o3.at[:, pl.ds(i * _CHUNK, _CHUNK), :],
            out_sems.at[b],
        )

    for i in range(nch):
        b = i % _NBUF
        if i >= _NBUF:
            out_copy(i - _NBUF, b).wait()
        in_copy(i, b).start(priority=i % 2)
        j = i - _LAG
        if j >= 0:
            bj = j % _NBUF
            in_copy(j, bj).wait()
            out_copy(j, bj).start(priority=j % 2)
    for j in range(max(0, nch - _LAG), nch):
        bj = j % _NBUF
        in_copy(j, bj).wait()
        out_copy(j, bj).start(priority=j % 2)
    for j in range(max(0, nch - _NBUF), nch):
        out_copy(j, j % _NBUF).wait()


def kernel(table):
    rows, cols = table.shape
    return pl.pallas_call(
        _copy_body,
        out_shape=jax.ShapeDtypeStruct(table.shape, table.dtype),
        in_specs=[pl.BlockSpec(memory_space=pl.ANY)],
        out_specs=pl.BlockSpec(memory_space=pl.ANY),
        scratch_shapes=[
            pltpu.VMEM((_NBUF, _OUTER, _CHUNK, cols), table.dtype),
            pltpu.SemaphoreType.DMA((_NBUF,)),
            pltpu.SemaphoreType.DMA((_NBUF,)),
        ],
    )(table)


# speedup vs baseline: 1.8195x; 1.3139x over previous
import jax
import jax.numpy as jnp
from jax.experimental import pallas as pl
from jax.experimental.pallas import tpu as pltpu

_OUTER = 125
_CHUNK = 16
_NBUF = 32
_LAG = 16


def _copy_body(x_ref, o_ref, buf, in_sems, out_sems):
    inner = x_ref.shape[1]
    nch = inner // _CHUNK

    def in_copy(i, b):
        return pltpu.make_async_copy(
            x_ref.at[:, pl.ds(i * _CHUNK, _CHUNK), :],
            buf.at[b],
            in_sems.at[b],
        )

    def out_copy(i, b):
        return pltpu.make_async_copy(
            buf.at[b],
            o_ref.at[:, pl.ds(i * _CHUNK, _CHUNK), :],
            out_sems.at[b],
        )

    for i in range(nch):
        b = i % _NBUF
        if i >= _NBUF:
            out_copy(i - _NBUF, b).wait()
        in_copy(i, b).start(priority=i % 2)
        j = i - _LAG
        if j >= 0:
            bj = j % _NBUF
            in_copy(j, bj).wait()
            out_copy(j, bj).start(priority=j % 2)
    for j in range(max(0, nch - _LAG), nch):
        bj = j % _NBUF
        in_copy(j, bj).wait()
        out_copy(j, bj).start(priority=j % 2)
    for j in range(max(0, nch - _NBUF), nch):
        out_copy(j, j % _NBUF).wait()


def kernel(table):
    rows, cols = table.shape
    inner = rows // _OUTER
    t = table.reshape(_OUTER, inner, cols)
    out = pl.pallas_call(
        _copy_body,
        out_shape=jax.ShapeDtypeStruct((_OUTER, inner, cols), table.dtype),
        in_specs=[pl.BlockSpec(memory_space=pl.ANY)],
        out_specs=pl.BlockSpec(memory_space=pl.ANY),
        scratch_shapes=[
            pltpu.VMEM((_NBUF, _OUTER, _CHUNK, cols), table.dtype),
            pltpu.SemaphoreType.DMA((_NBUF,)),
            pltpu.SemaphoreType.DMA((_NBUF,)),
        ],
    )(t)
    return out.reshape(rows, cols)


# speedup vs baseline: 1.8213x; 1.0010x over previous
import jax
import jax.numpy as jnp
from jax.experimental import pallas as pl
from jax.experimental.pallas import tpu as pltpu

_OUTER = 500
_CHUNK = 4
_NBUF = 32
_LAG = 16


def _copy_body(x_ref, o_ref, buf, in_sems, out_sems):
    inner = x_ref.shape[1]
    nch = inner // _CHUNK

    def in_copy(i, b):
        return pltpu.make_async_copy(
            x_ref.at[:, pl.ds(i * _CHUNK, _CHUNK), :],
            buf.at[b],
            in_sems.at[b],
        )

    def out_copy(i, b):
        return pltpu.make_async_copy(
            buf.at[b],
            o_ref.at[:, pl.ds(i * _CHUNK, _CHUNK), :],
            out_sems.at[b],
        )

    for i in range(nch):
        b = i % _NBUF
        if i >= _NBUF:
            out_copy(i - _NBUF, b).wait()
        in_copy(i, b).start(priority=i % 2)
        j = i - _LAG
        if j >= 0:
            bj = j % _NBUF
            in_copy(j, bj).wait()
            out_copy(j, bj).start(priority=j % 2)
    for j in range(max(0, nch - _LAG), nch):
        bj = j % _NBUF
        in_copy(j, bj).wait()
        out_copy(j, bj).start(priority=j % 2)
    for j in range(max(0, nch - _NBUF), nch):
        out_copy(j, j % _NBUF).wait()


def kernel(table):
    rows, cols = table.shape
    inner = rows // _OUTER
    t = table.reshape(_OUTER, inner, cols)
    out = pl.pallas_call(
        _copy_body,
        out_shape=jax.ShapeDtypeStruct((_OUTER, inner, cols), table.dtype),
        in_specs=[pl.BlockSpec(memory_space=pl.ANY)],
        out_specs=pl.BlockSpec(memory_space=pl.ANY),
        scratch_shapes=[
            pltpu.VMEM((_NBUF, _OUTER, _CHUNK, cols), table.dtype),
            pltpu.SemaphoreType.DMA((_NBUF,)),
            pltpu.SemaphoreType.DMA((_NBUF,)),
        ],
    )(t)
    return out.reshape(rows, cols)
